# baseline (speedup 1.0000x reference)
import functools

import jax
import jax.numpy as jnp
from jax.experimental import pallas as pl
from jax.experimental.pallas import tpu as pltpu

_EPS = 1e-5


def _fold_h(v, op):
    h = v.shape[1]
    while h > 8:
        h //= 2
        v = op(v[:, :h, :], v[:, h:, :])
    while h > 1:
        h //= 2
        v = op(v[:, :h, :], v[:, h:, :])
    return v


def _mam_kernel(inp_ref, x_ref, w1b_ref, w2b_ref, out_ref, *, bc, h, w, inv_hw):
    x = x_ref[...].reshape(bc, h, w)

    h2 = h // 2
    sslab = 4
    s_p, ss_p, mx_p = [], [], []
    for c0 in range(0, bc, sslab):
        xs = x[c0:c0 + sslab]
        xlo = xs[:, :h2, :]
        xhi = xs[:, h2:, :]
        s_p.append(jnp.sum(_fold_h(xlo + xhi, jnp.add),
                           axis=2, keepdims=True))
        ss_p.append(jnp.sum(_fold_h(xlo * xlo + xhi * xhi, jnp.add),
                            axis=2, keepdims=True))
        mx_p.append(jnp.max(_fold_h(jnp.maximum(xlo, xhi), jnp.maximum),
                            axis=2, keepdims=True))
    s = jnp.concatenate(s_p, axis=0)
    ss = jnp.concatenate(ss_p, axis=0)
    mx = jnp.concatenate(mx_p, axis=0)

    mu = (s * inv_hw).reshape(bc, 1)
    var = jnp.maximum(ss.reshape(bc, 1) * inv_hw - mu * mu, 0.0)
    rstd = jax.lax.rsqrt(var + _EPS)

    lane2 = jax.lax.broadcasted_iota(jnp.int32, (bc, 2), 1)
    p2 = jnp.where(lane2 == 0, mu, mx.reshape(bc, 1))
    hh = jnp.maximum(
        jnp.dot(w1b_ref[...], p2, preferred_element_type=jnp.float32), 0.0)
    m = jax.nn.sigmoid(
        jnp.dot(w2b_ref[...], hh, preferred_element_type=jnp.float32))
    mask = m[:, 0:1] + m[:, 1:2]

    a = rstd * (1.0 - mask)
    b = -mu * a
    mask3 = mask[:, :, None]
    a3 = a[:, :, None]
    b3 = b[:, :, None]
    slab = 1
    inp3 = inp_ref[...].reshape(bc, h, w)
    for c0 in range(0, bc, slab):
        sl = slice(c0, c0 + slab)
        out = (inp3[sl] * mask3[sl] + x[sl] * a3[sl] + b3[sl])
        out_ref[c0 * h:(c0 + slab) * h, :] = out.reshape(slab * h, w)


def kernel(inp, x, w1, w2):
    N, C, H, W = x.shape
    HW = H * W
    Cr = w1.shape[0]

    B = 2 if N % 2 == 0 else 1
    G = N // B
    BC = B * C

    inp_f = inp.reshape(N * C * H, W)
    x_f = x.reshape(N * C * H, W)
    eye = jnp.eye(B, dtype=jnp.float32)
    w1b = jnp.kron(eye, w1.astype(jnp.float32))
    w2b = jnp.kron(eye, w2.astype(jnp.float32))

    out = pl.pallas_call(
        functools.partial(_mam_kernel, bc=BC, h=H, w=W, inv_hw=float(1.0 / HW)),
        out_shape=jax.ShapeDtypeStruct((N * C * H, W), x.dtype),
        grid=(G,),
        in_specs=[
            pl.BlockSpec((BC * H, W), lambda g: (g, 0)),
            pl.BlockSpec((BC * H, W), lambda g: (g, 0)),
            pl.BlockSpec((B * Cr, BC), lambda g: (0, 0)),
            pl.BlockSpec((BC, B * Cr), lambda g: (0, 0)),
        ],
        out_specs=pl.BlockSpec((BC * H, W), lambda g: (g, 0)),
        compiler_params=pltpu.CompilerParams(
            dimension_semantics=("parallel",),
            vmem_limit_bytes=60 << 20,
        ),
        cost_estimate=pl.CostEstimate(
            flops=int(N * (10 * C * HW + 8 * Cr * C)),
            transcendentals=int(N * 3 * C),
            bytes_accessed=int(3 * N * C * HW * 4),
        ),
    )(inp_f, x_f, w1b, w2b)
    return out.reshape(N, C, H, W)


# speedup vs baseline: 1.0556x; 1.0556x over previous
import functools

import jax
import jax.numpy as jnp
from jax.experimental import pallas as pl
from jax.experimental.pallas import tpu as pltpu

_EPS = 1e-5


def _fold_h(v, op):
    h = v.shape[1]
    while h > 8:
        h //= 2
        v = op(v[:, :h, :], v[:, h:, :])
    while h > 1:
        h //= 2
        v = op(v[:, :h, :], v[:, h:, :])
    return v


def _mam_kernel(inp_ref, x_ref, w1b_ref, w2b_ref, out_ref, *, bc, h, w, inv_hw):
    x = x_ref[...].reshape(bc, h, w)

    h2 = h // 2
    xlo = x[:, :h2, :]
    xhi = x[:, h2:, :]
    s = jnp.sum(_fold_h(xlo + xhi, jnp.add), axis=2, keepdims=True)
    ss = jnp.sum(_fold_h(xlo * xlo + xhi * xhi, jnp.add),
                 axis=2, keepdims=True)
    mx = jnp.max(_fold_h(jnp.maximum(xlo, xhi), jnp.maximum),
                 axis=2, keepdims=True)

    mu = (s * inv_hw).reshape(bc, 1)
    var = jnp.maximum(ss.reshape(bc, 1) * inv_hw - mu * mu, 0.0)
    rstd = jax.lax.rsqrt(var + _EPS)

    lane2 = jax.lax.broadcasted_iota(jnp.int32, (bc, 2), 1)
    p2 = jnp.where(lane2 == 0, mu, mx.reshape(bc, 1))
    hh = jnp.maximum(
        jnp.dot(w1b_ref[...], p2, preferred_element_type=jnp.float32), 0.0)
    m = jax.nn.sigmoid(
        jnp.dot(w2b_ref[...], hh, preferred_element_type=jnp.float32))
    mask = m[:, 0:1] + m[:, 1:2]

    a = rstd * (1.0 - mask)
    b = -mu * a
    mask3 = mask[:, :, None]
    a3 = a[:, :, None]
    b3 = b[:, :, None]
    slab = 1
    inp3 = inp_ref[...].reshape(bc, h, w)
    for c0 in range(0, bc, slab):
        sl = slice(c0, c0 + slab)
        out = (inp3[sl] * mask3[sl] + x[sl] * a3[sl] + b3[sl])
        out_ref[c0 * h:(c0 + slab) * h, :] = out.reshape(slab * h, w)


def kernel(inp, x, w1, w2):
    N, C, H, W = x.shape
    HW = H * W
    Cr = w1.shape[0]

    B = 4 if N % 4 == 0 else 1
    G = N // B
    BC = B * C

    inp_f = inp.reshape(N * C * H, W)
    x_f = x.reshape(N * C * H, W)
    eye = jnp.eye(B, dtype=jnp.float32)
    w1b = jnp.kron(eye, w1.astype(jnp.float32))
    w2b = jnp.kron(eye, w2.astype(jnp.float32))

    out = pl.pallas_call(
        functools.partial(_mam_kernel, bc=BC, h=H, w=W, inv_hw=float(1.0 / HW)),
        out_shape=jax.ShapeDtypeStruct((N * C * H, W), x.dtype),
        grid=(G,),
        in_specs=[
            pl.BlockSpec((BC * H, W), lambda g: (g, 0)),
            pl.BlockSpec((BC * H, W), lambda g: (g, 0)),
            pl.BlockSpec((B * Cr, BC), lambda g: (0, 0)),
            pl.BlockSpec((BC, B * Cr), lambda g: (0, 0)),
        ],
        out_specs=pl.BlockSpec((BC * H, W), lambda g: (g, 0)),
        compiler_params=pltpu.CompilerParams(
            dimension_semantics=("parallel",),
            vmem_limit_bytes=60 << 20,
        ),
        cost_estimate=pl.CostEstimate(
            flops=int(N * (10 * C * HW + 8 * Cr * C)),
            transcendentals=int(N * 3 * C),
            bytes_accessed=int(3 * N * C * HW * 4),
        ),
    )(inp_f, x_f, w1b, w2b)
    return out.reshape(N, C, H, W)
